# baseline (speedup 1.0000x reference)
import jax
import jax.numpy as jnp
from jax import lax
from jax.experimental import pallas as pl

B = 8
N = 116
E = 2048
IN_V = 116
OUT_V = 64
IN_E = 16


def _body(hv_ref, he_ref, av_ref, t_ref, w_ref, pt_ref, b_ref, out_ref):
    w = w_ref[...].astype(jnp.bfloat16)
    pt = pt_ref[0].astype(jnp.bfloat16)
    bias2 = b_ref[...]
    row = lax.broadcasted_iota(jnp.int32, (N, N), 0)
    col = lax.broadcasted_iota(jnp.int32, (N, N), 1)
    for b in range(B):
        t = t_ref[b]
        d_col = jnp.dot(he_ref[b].astype(jnp.bfloat16), pt,
                        preferred_element_type=jnp.float32)
        d = jnp.transpose(d_col)
        td = (t * d).astype(jnp.bfloat16)
        s = lax.dot_general(td, t.astype(jnp.bfloat16), (((1,), (1,)), ((), ())),
                            preferred_element_type=jnp.float32)
        m1 = jnp.where(row == col, 1.0, s)
        a = (m1 * av_ref[b]).astype(jnp.bfloat16)
        x = jnp.dot(hv_ref[b].astype(jnp.bfloat16), w,
                    preferred_element_type=jnp.float32)
        out_ref[b] = jnp.dot(a, x.astype(jnp.bfloat16),
                             preferred_element_type=jnp.float32) + bias2


def kernel(H_v, H_e, adj_e, adj_v, T, weight, p, bias):
    del adj_e
    hv = H_v.reshape(B, N, IN_V)
    he = H_e.reshape(B, E, IN_E)
    av = adj_v.reshape(B, N, N)
    t = T.reshape(B, N, E)
    pt = p.reshape(1, IN_E, 1)
    b2 = bias.reshape(1, OUT_V)

    out = pl.pallas_call(
        _body,
        in_specs=[
            pl.BlockSpec((B, N, IN_V), lambda: (0, 0, 0)),
            pl.BlockSpec((B, E, IN_E), lambda: (0, 0, 0)),
            pl.BlockSpec((B, N, N), lambda: (0, 0, 0)),
            pl.BlockSpec((B, N, E), lambda: (0, 0, 0)),
            pl.BlockSpec((IN_V, OUT_V), lambda: (0, 0)),
            pl.BlockSpec((1, IN_E, 1), lambda: (0, 0, 0)),
            pl.BlockSpec((1, OUT_V), lambda: (0, 0)),
        ],
        out_specs=pl.BlockSpec((B, N, OUT_V), lambda: (0, 0, 0)),
        out_shape=jax.ShapeDtypeStruct((B, N, OUT_V), jnp.float32),
    )(hv, he, av, t, weight, pt, b2)

    return (out, he)


# speedup vs baseline: 1.0316x; 1.0316x over previous
import jax
import jax.numpy as jnp
from jax import lax
from jax.experimental import pallas as pl

B = 8
N = 116
E = 2048
IN_V = 116
OUT_V = 64
IN_E = 16


def _body(hv_ref, he_ref, av_ref, t_ref, w_ref, pt_ref, b_ref, out_ref):
    w = w_ref[...].astype(jnp.bfloat16)
    pv = pt_ref[0, :, 0]
    bias2 = b_ref[...]
    row = lax.broadcasted_iota(jnp.int32, (N, N), 0)
    col = lax.broadcasted_iota(jnp.int32, (N, N), 1)
    for b in range(B):
        t = t_ref[b]
        d = jnp.dot(he_ref[b], pv,
                    preferred_element_type=jnp.float32)
        td = (t * d[None, :]).astype(jnp.bfloat16)
        s = lax.dot_general(td, t.astype(jnp.bfloat16), (((1,), (1,)), ((), ())),
                            preferred_element_type=jnp.float32)
        m1 = jnp.where(row == col, 1.0, s)
        a = (m1 * av_ref[b]).astype(jnp.bfloat16)
        x = jnp.dot(hv_ref[b].astype(jnp.bfloat16), w,
                    preferred_element_type=jnp.float32)
        out_ref[b] = jnp.dot(a, x.astype(jnp.bfloat16),
                             preferred_element_type=jnp.float32) + bias2


def kernel(H_v, H_e, adj_e, adj_v, T, weight, p, bias):
    del adj_e
    hv = H_v.reshape(B, N, IN_V)
    he = H_e.reshape(B, E, IN_E)
    av = adj_v.reshape(B, N, N)
    t = T.reshape(B, N, E)
    pt = p.reshape(1, IN_E, 1)
    b2 = bias.reshape(1, OUT_V)

    out = pl.pallas_call(
        _body,
        in_specs=[
            pl.BlockSpec((B, N, IN_V), lambda: (0, 0, 0)),
            pl.BlockSpec((B, E, IN_E), lambda: (0, 0, 0)),
            pl.BlockSpec((B, N, N), lambda: (0, 0, 0)),
            pl.BlockSpec((B, N, E), lambda: (0, 0, 0)),
            pl.BlockSpec((IN_V, OUT_V), lambda: (0, 0)),
            pl.BlockSpec((1, IN_E, 1), lambda: (0, 0, 0)),
            pl.BlockSpec((1, OUT_V), lambda: (0, 0)),
        ],
        out_specs=pl.BlockSpec((B, N, OUT_V), lambda: (0, 0, 0)),
        out_shape=jax.ShapeDtypeStruct((B, N, OUT_V), jnp.float32),
    )(hv, he, av, t, weight, pt, b2)

    return (out, he)


# speedup vs baseline: 1.6202x; 1.5705x over previous
import jax
import jax.numpy as jnp
from jax import lax
from jax.experimental import pallas as pl

B = 8
N = 116
E = 2048
IN_V = 116
OUT_V = 64
IN_E = 16


def _body(hv_ref, he_ref, av_ref, t_ref, w_ref, p_ref, b_ref, out_ref):
    w = w_ref[...].astype(jnp.bfloat16)
    p2 = p_ref[...].astype(jnp.bfloat16)
    bias2 = b_ref[...]
    row = lax.broadcasted_iota(jnp.int32, (N, N), 0)
    col = lax.broadcasted_iota(jnp.int32, (N, N), 1)
    for b in range(B):
        t = t_ref[pl.ds(b * N, N), :]
        he_b = he_ref[pl.ds(b * E, E), :]
        d = lax.dot_general(p2, he_b.astype(jnp.bfloat16), (((1,), (1,)), ((), ())),
                            preferred_element_type=jnp.float32)
        td = (t * d).astype(jnp.bfloat16)
        s = lax.dot_general(td, t.astype(jnp.bfloat16), (((1,), (1,)), ((), ())),
                            preferred_element_type=jnp.float32)
        m1 = jnp.where(row == col, 1.0, s)
        a = (m1 * av_ref[pl.ds(b * N, N), :]).astype(jnp.bfloat16)
        x = jnp.dot(hv_ref[pl.ds(b * N, N), :].astype(jnp.bfloat16), w,
                    preferred_element_type=jnp.float32)
        out_ref[b] = jnp.dot(a, x.astype(jnp.bfloat16),
                             preferred_element_type=jnp.float32) + bias2


def kernel(H_v, H_e, adj_e, adj_v, T, weight, p, bias):
    del adj_e
    b2 = bias.reshape(1, OUT_V)

    out = pl.pallas_call(
        _body,
        in_specs=[
            pl.BlockSpec((B * N, IN_V), lambda: (0, 0)),
            pl.BlockSpec((B * E, IN_E), lambda: (0, 0)),
            pl.BlockSpec((B * N, N), lambda: (0, 0)),
            pl.BlockSpec((B * N, E), lambda: (0, 0)),
            pl.BlockSpec((IN_V, OUT_V), lambda: (0, 0)),
            pl.BlockSpec((1, IN_E), lambda: (0, 0)),
            pl.BlockSpec((1, OUT_V), lambda: (0, 0)),
        ],
        out_specs=pl.BlockSpec((B, N, OUT_V), lambda: (0, 0, 0)),
        out_shape=jax.ShapeDtypeStruct((B, N, OUT_V), jnp.float32),
    )(H_v, H_e, adj_v, T, weight, p, b2)

    return (out, H_e.reshape(B, E, IN_E))
